# Initial kernel scaffold; baseline (speedup 1.0000x reference)
import jax
import jax.numpy as jnp
from jax.experimental import pallas as pl


def kernel(state, feature, edge_dist, W1, b1, W2, b2, W3, b3, gate, edge_src, edge_dst):
    raise NotImplementedError("write your pallas kernel here")


# speedup vs baseline: 10.7410x; 10.7410x over previous
import functools

import jax
import jax.numpy as jnp
from jax import lax
from jax.experimental import pallas as pl
from jax.experimental.pallas import tpu as pltpu
from jax.experimental.pallas import tpu_sc as plsc

N = 1024
K = 8
E = N * K
H = 32
B = 4
T = 12
BT = B * T
D = BT * H
BLK = 128
NBE = E // BLK
FP = 128


def _sc_gather(s, feature, edge_src, edge_dst):
    info = plsc.get_sparse_core_info()
    nw = info.num_cores * info.num_subcores
    per_w = E // nw
    C = 32
    n_chunks = per_w // C
    mesh = plsc.VectorSubcoreMesh(core_axis_name="c", subcore_axis_name="s")

    @functools.partial(
        pl.kernel,
        mesh=mesh,
        out_type=[
            jax.ShapeDtypeStruct((E, D), jnp.float32),
            jax.ShapeDtypeStruct((E, D), jnp.float32),
            jax.ShapeDtypeStruct((E, FP), jnp.float32),
            jax.ShapeDtypeStruct((E, FP), jnp.float32),
        ],
        scratch_types=[
            pltpu.VMEM((C,), jnp.int32),
            pltpu.VMEM((C, D), jnp.float32),
            pltpu.VMEM((C, FP), jnp.float32),
            pltpu.SemaphoreType.DMA,
        ],
    )
    def k(s_hbm, f_hbm, src_hbm, dst_hbm, o_ss, o_ds, o_fs, o_fd,
          idx_v, rows_v, frows_v, sem):
        wid = lax.axis_index("s") * info.num_cores + lax.axis_index("c")
        base = wid * per_w

        def chunk(g, carry):
            off = base + g * C
            pltpu.sync_copy(src_hbm.at[pl.ds(off, C)], idx_v)
            pltpu.async_copy(s_hbm.at[idx_v], rows_v, sem).wait()
            pltpu.sync_copy(rows_v, o_ss.at[pl.ds(off, C)])
            pltpu.async_copy(f_hbm.at[idx_v], frows_v, sem).wait()
            pltpu.sync_copy(frows_v, o_fs.at[pl.ds(off, C)])
            pltpu.sync_copy(dst_hbm.at[pl.ds(off, C)], idx_v)
            pltpu.async_copy(s_hbm.at[idx_v], rows_v, sem).wait()
            pltpu.sync_copy(rows_v, o_ds.at[pl.ds(off, C)])
            pltpu.async_copy(f_hbm.at[idx_v], frows_v, sem).wait()
            pltpu.sync_copy(frows_v, o_fd.at[pl.ds(off, C)])
            return carry

        lax.fori_loop(0, n_chunks, chunk, 0)

    return k(s, feature, edge_src, edge_dst)


def _tc_body(seg_dst_ref, nseg_ref, src_ref, dst_ref, fsrc_ref, fdst_ref,
             dist_ref, lsi_ref, w1_ref, b1_ref, w2_ref, b2_ref, w3_ref,
             b3_ref, sg_ref, out_ref, num_ref, den_ref, part_ref):
    b = pl.program_id(0)

    @pl.when(b == 0)
    def _init():
        num_ref[...] = jnp.zeros_like(num_ref)
        den_ref[...] = jnp.zeros_like(den_ref)

    x = jnp.concatenate(
        [fsrc_ref[:, :H], fdst_ref[:, :H], dist_ref[...]], axis=1)
    h = 1.0 / (1.0 + jnp.exp(-(jnp.dot(x, w1_ref[...],
                                       preferred_element_type=jnp.float32)
                               + b1_ref[...])))
    h = 1.0 / (1.0 + jnp.exp(-(jnp.dot(h, w2_ref[...],
                                       preferred_element_type=jnp.float32)
                               + b2_ref[...])))
    w = jnp.dot(h, w3_ref[...],
                preferred_element_type=jnp.float32) + b3_ref[...]

    src3 = src_ref[...].reshape(BLK, BT, H)
    dst3 = dst_ref[...].reshape(BLK, BT, H)
    wt = w[:, :H * H].reshape(BLK, H, H)
    wb = w[:, H * H:].reshape(BLK, H, H)
    dn = (((2,), (1,)), ((0,), (0,)))
    alpha = (lax.dot_general(src3, wt, dn, preferred_element_type=jnp.float32)
             + lax.dot_general(dst3, wb, dn,
                               preferred_element_type=jnp.float32))
    alpha = jnp.where(alpha >= 0, alpha, 0.01 * alpha)
    ex = jnp.exp(jnp.minimum(alpha, 80.0))

    payload = jnp.concatenate(
        [(ex * src3).reshape(BLK, D), ex.reshape(BLK, D)], axis=1)

    lsi = jnp.broadcast_to(lsi_ref[0], (BLK, BLK))
    pt = (lax.broadcasted_iota(jnp.int32, (BLK, BLK), 0) == lsi)
    part_ref[...] = jnp.dot(pt.astype(jnp.float32), payload,
                            preferred_element_type=jnp.float32)

    ns = nseg_ref[b]

    def scatter(j, carry):
        d = seg_dst_ref[b * BLK + j]
        num_ref[pl.ds(d, 1), :] = (num_ref[pl.ds(d, 1), :]
                                   + part_ref[pl.ds(j, 1), :D])
        den_ref[pl.ds(d, 1), :] = (den_ref[pl.ds(d, 1), :]
                                   + part_ref[pl.ds(j, 1), D:])
        return carry

    lax.fori_loop(0, ns, scatter, 0)

    @pl.when(b == NBE - 1)
    def _final():
        den = den_ref[...]
        dsafe = jnp.where(den > 0, den, 1.0)
        out_ref[...] = jnp.maximum(num_ref[...] / dsafe, 0.0) * sg_ref[0, 0]


def _tc_compute(src_st, dst_st, fsrc, fdst, edge_dist, lsi, seg_dst, nseg,
                W1, b1, W2, b2, W3, b3, sg):
    grid_spec = pltpu.PrefetchScalarGridSpec(
        num_scalar_prefetch=2,
        grid=(NBE,),
        in_specs=[
            pl.BlockSpec((BLK, D), lambda b, *_: (b, 0)),
            pl.BlockSpec((BLK, D), lambda b, *_: (b, 0)),
            pl.BlockSpec((BLK, FP), lambda b, *_: (b, 0)),
            pl.BlockSpec((BLK, FP), lambda b, *_: (b, 0)),
            pl.BlockSpec((BLK, 1), lambda b, *_: (b, 0)),
            pl.BlockSpec((1, 1, BLK), lambda b, *_: (b, 0, 0)),
            pl.BlockSpec((2 * H + 1, 32), lambda b, *_: (0, 0)),
            pl.BlockSpec((1, 32), lambda b, *_: (0, 0)),
            pl.BlockSpec((32, 16), lambda b, *_: (0, 0)),
            pl.BlockSpec((1, 16), lambda b, *_: (0, 0)),
            pl.BlockSpec((16, 2 * H * H), lambda b, *_: (0, 0)),
            pl.BlockSpec((1, 2 * H * H), lambda b, *_: (0, 0)),
            pl.BlockSpec(memory_space=pltpu.SMEM),
        ],
        out_specs=pl.BlockSpec((N, D), lambda b, *_: (0, 0)),
        scratch_shapes=[
            pltpu.VMEM((N, D), jnp.float32),
            pltpu.VMEM((N, D), jnp.float32),
            pltpu.VMEM((BLK, 2 * D), jnp.float32),
        ],
    )
    return pl.pallas_call(
        _tc_body,
        grid_spec=grid_spec,
        out_shape=jax.ShapeDtypeStruct((N, D), jnp.float32),
    )(seg_dst, nseg, src_st, dst_st, fsrc, fdst, edge_dist, lsi,
      W1, b1, W2, b2, W3, b3, sg)


def kernel(state, feature, edge_dist, W1, b1, W2, b2, W3, b3, gate,
           edge_src, edge_dst):
    s = jnp.transpose(state, (2, 0, 1, 3)).reshape(N, D)

    feature_p = jnp.pad(feature, ((0, 0), (0, FP - H)))
    src_st, dst_st, fsrc, fdst = _sc_gather(s, feature_p, edge_src, edge_dst)

    prev = jnp.concatenate([jnp.full((1,), -1, jnp.int32), edge_dst[:-1]])
    pos = jnp.arange(E, dtype=jnp.int32) % BLK
    is_start = (edge_dst != prev) | (pos == 0)
    lsi = jnp.cumsum(is_start.reshape(NBE, BLK).astype(jnp.int32),
                     axis=1) - 1
    nseg = lsi[:, -1] + 1
    seg_dst = jnp.zeros((NBE, BLK), jnp.int32).at[
        jnp.arange(E, dtype=jnp.int32) // BLK, lsi.reshape(-1)
    ].set(edge_dst).reshape(-1)

    out2d = _tc_compute(
        src_st, dst_st, fsrc, fdst, edge_dist,
        lsi.reshape(NBE, 1, BLK), seg_dst, nseg,
        W1, b1.reshape(1, -1), W2, b2.reshape(1, -1), W3,
        b3.reshape(1, -1), jax.nn.sigmoid(gate))

    return jnp.transpose(out2d.reshape(N, B, T, H), (1, 2, 0, 3))


# speedup vs baseline: 11.9972x; 1.1170x over previous
import functools

import jax
import jax.numpy as jnp
from jax import lax
from jax.experimental import pallas as pl
from jax.experimental.pallas import tpu as pltpu
from jax.experimental.pallas import tpu_sc as plsc

N = 1024
K = 8
E = N * K
H = 32
B = 4
T = 12
BT = B * T
D = BT * H
BLK = 128
NBE = E // BLK
FP = 128


def _sc_gather(s, feature, edge_src, edge_dst):
    info = plsc.get_sparse_core_info()
    nw = info.num_cores * info.num_subcores
    per_w = E // nw
    C = 32
    n_chunks = per_w // C
    mesh = plsc.VectorSubcoreMesh(core_axis_name="c", subcore_axis_name="s")

    @functools.partial(
        pl.kernel,
        mesh=mesh,
        out_type=[
            jax.ShapeDtypeStruct((E, D), jnp.float32),
            jax.ShapeDtypeStruct((E, D), jnp.float32),
            jax.ShapeDtypeStruct((E, FP), jnp.float32),
            jax.ShapeDtypeStruct((E, FP), jnp.float32),
        ],
        scratch_types=[
            pltpu.VMEM((C,), jnp.int32),
            pltpu.VMEM((C, D), jnp.float32),
            pltpu.VMEM((C, FP), jnp.float32),
            pltpu.SemaphoreType.DMA,
        ],
    )
    def k(s_hbm, f_hbm, src_hbm, dst_hbm, o_ss, o_ds, o_fs, o_fd,
          idx_v, rows_v, frows_v, sem):
        wid = lax.axis_index("s") * info.num_cores + lax.axis_index("c")
        base = wid * per_w

        def chunk(g, carry):
            off = base + g * C
            pltpu.sync_copy(src_hbm.at[pl.ds(off, C)], idx_v)
            pltpu.async_copy(s_hbm.at[idx_v], rows_v, sem).wait()
            pltpu.sync_copy(rows_v, o_ss.at[pl.ds(off, C)])
            pltpu.async_copy(f_hbm.at[idx_v], frows_v, sem).wait()
            pltpu.sync_copy(frows_v, o_fs.at[pl.ds(off, C)])
            pltpu.sync_copy(dst_hbm.at[pl.ds(off, C)], idx_v)
            pltpu.async_copy(s_hbm.at[idx_v], rows_v, sem).wait()
            pltpu.sync_copy(rows_v, o_ds.at[pl.ds(off, C)])
            pltpu.async_copy(f_hbm.at[idx_v], frows_v, sem).wait()
            pltpu.sync_copy(frows_v, o_fd.at[pl.ds(off, C)])
            return carry

        lax.fori_loop(0, n_chunks, chunk, 0)

    return k(s, feature, edge_src, edge_dst)


def _tc_body(seg_dst_ref, nseg_ref, src_ref, dst_ref, fsrc_ref, fdst_ref,
             dist_ref, lsi_ref, w1_ref, b1_ref, w2_ref, b2_ref, w3_ref,
             b3_ref, sg_ref, out_ref, num_ref, den_ref, part_ref):
    b = pl.program_id(0)

    @pl.when(b == 0)
    def _init():
        num_ref[...] = jnp.zeros_like(num_ref)
        den_ref[...] = jnp.zeros_like(den_ref)

    x = jnp.concatenate(
        [fsrc_ref[:, :H], fdst_ref[:, :H], dist_ref[...]], axis=1)
    h = 1.0 / (1.0 + jnp.exp(-(jnp.dot(x, w1_ref[...],
                                       preferred_element_type=jnp.float32)
                               + b1_ref[...])))
    h = 1.0 / (1.0 + jnp.exp(-(jnp.dot(h, w2_ref[...],
                                       preferred_element_type=jnp.float32)
                               + b2_ref[...])))
    w3d = lax.dot_general(h, w3_ref[...], (((1,), (0,)), ((), ())),
                          preferred_element_type=jnp.float32) + b3_ref[...]

    src_flat = src_ref[...]
    src3 = src_flat.reshape(BLK, BT, H)
    dst3 = dst_ref[...].reshape(BLK, BT, H)
    wt = w3d[:, :H, :]
    wb = w3d[:, H:, :]
    dn = (((2,), (1,)), ((0,), (0,)))
    alpha = (lax.dot_general(src3, wt, dn, preferred_element_type=jnp.float32)
             + lax.dot_general(dst3, wb, dn,
                               preferred_element_type=jnp.float32))
    alpha = jnp.where(alpha >= 0, alpha, 0.01 * alpha)
    ex = jnp.exp(jnp.minimum(alpha, 80.0)).reshape(BLK, D)

    payload = jnp.concatenate([ex * src_flat, ex], axis=1)

    lsi = jnp.broadcast_to(lsi_ref[0], (BLK, BLK))
    pt = (lax.broadcasted_iota(jnp.int32, (BLK, BLK), 0) == lsi)
    part_ref[...] = jnp.dot(pt.astype(jnp.float32), payload,
                            preferred_element_type=jnp.float32)

    ns = nseg_ref[b]

    def scatter(j, carry):
        d = seg_dst_ref[b * BLK + j]
        num_ref[pl.ds(d, 1), :] = (num_ref[pl.ds(d, 1), :]
                                   + part_ref[pl.ds(j, 1), :D])
        den_ref[pl.ds(d, 1), :] = (den_ref[pl.ds(d, 1), :]
                                   + part_ref[pl.ds(j, 1), D:])
        return carry

    lax.fori_loop(0, ns, scatter, 0)

    @pl.when(b == NBE - 1)
    def _final():
        den = den_ref[...]
        dsafe = jnp.where(den > 0, den, 1.0)
        out_ref[...] = jnp.maximum(num_ref[...] / dsafe, 0.0) * sg_ref[0, 0]


def _tc_compute(src_st, dst_st, fsrc, fdst, edge_dist, lsi, seg_dst, nseg,
                W1, b1, W2, b2, W3, b3, sg):
    grid_spec = pltpu.PrefetchScalarGridSpec(
        num_scalar_prefetch=2,
        grid=(NBE,),
        in_specs=[
            pl.BlockSpec((BLK, D), lambda b, *_: (b, 0)),
            pl.BlockSpec((BLK, D), lambda b, *_: (b, 0)),
            pl.BlockSpec((BLK, FP), lambda b, *_: (b, 0)),
            pl.BlockSpec((BLK, FP), lambda b, *_: (b, 0)),
            pl.BlockSpec((BLK, 1), lambda b, *_: (b, 0)),
            pl.BlockSpec((1, 1, BLK), lambda b, *_: (b, 0, 0)),
            pl.BlockSpec((2 * H + 1, 32), lambda b, *_: (0, 0)),
            pl.BlockSpec((1, 32), lambda b, *_: (0, 0)),
            pl.BlockSpec((32, 16), lambda b, *_: (0, 0)),
            pl.BlockSpec((1, 16), lambda b, *_: (0, 0)),
            pl.BlockSpec((16, 2 * H, H), lambda b, *_: (0, 0, 0)),
            pl.BlockSpec((1, 2 * H, H), lambda b, *_: (0, 0, 0)),
            pl.BlockSpec(memory_space=pltpu.SMEM),
        ],
        out_specs=pl.BlockSpec((N, D), lambda b, *_: (0, 0)),
        scratch_shapes=[
            pltpu.VMEM((N, D), jnp.float32),
            pltpu.VMEM((N, D), jnp.float32),
            pltpu.VMEM((BLK, 2 * D), jnp.float32),
        ],
    )
    return pl.pallas_call(
        _tc_body,
        grid_spec=grid_spec,
        out_shape=jax.ShapeDtypeStruct((N, D), jnp.float32),
    )(seg_dst, nseg, src_st, dst_st, fsrc, fdst, edge_dist, lsi,
      W1, b1, W2, b2, W3.reshape(16, 2 * H, H), b3.reshape(1, 2 * H, H), sg)


def kernel(state, feature, edge_dist, W1, b1, W2, b2, W3, b3, gate,
           edge_src, edge_dst):
    s = jnp.transpose(state, (2, 0, 1, 3)).reshape(N, D)

    feature_p = jnp.pad(feature, ((0, 0), (0, FP - H)))
    src_st, dst_st, fsrc, fdst = _sc_gather(s, feature_p, edge_src, edge_dst)

    prev = jnp.concatenate([jnp.full((1,), -1, jnp.int32), edge_dst[:-1]])
    pos = jnp.arange(E, dtype=jnp.int32) % BLK
    is_start = (edge_dst != prev) | (pos == 0)
    lsi = jnp.cumsum(is_start.reshape(NBE, BLK).astype(jnp.int32),
                     axis=1) - 1
    nseg = lsi[:, -1] + 1
    seg_dst = jnp.zeros((NBE, BLK), jnp.int32).at[
        jnp.arange(E, dtype=jnp.int32) // BLK, lsi.reshape(-1)
    ].set(edge_dst).reshape(-1)

    out2d = _tc_compute(
        src_st, dst_st, fsrc, fdst, edge_dist,
        lsi.reshape(NBE, 1, BLK), seg_dst, nseg,
        W1, b1.reshape(1, -1), W2, b2.reshape(1, -1), W3,
        b3.reshape(1, -1), jax.nn.sigmoid(gate))

    return jnp.transpose(out2d.reshape(N, B, T, H), (1, 2, 0, 3))


# speedup vs baseline: 12.7621x; 1.0638x over previous
import functools

import jax
import jax.numpy as jnp
from jax import lax
from jax.experimental import pallas as pl
from jax.experimental.pallas import tpu as pltpu
from jax.experimental.pallas import tpu_sc as plsc

N = 1024
K = 8
E = N * K
H = 32
B = 4
T = 12
BT = B * T
D = BT * H
BLK = 128
NBE = E // BLK
FP = 128


def _sc_gather(s, feature, edge_src, edge_dst):
    info = plsc.get_sparse_core_info()
    nw = info.num_cores * info.num_subcores
    per_w = E // nw
    C = 32
    n_chunks = per_w // C
    mesh = plsc.VectorSubcoreMesh(core_axis_name="c", subcore_axis_name="s")

    @functools.partial(
        pl.kernel,
        mesh=mesh,
        out_type=[
            jax.ShapeDtypeStruct((E, D), jnp.float32),
            jax.ShapeDtypeStruct((E, D), jnp.float32),
            jax.ShapeDtypeStruct((E, FP), jnp.float32),
            jax.ShapeDtypeStruct((E, FP), jnp.float32),
        ],
        scratch_types=[
            pltpu.VMEM((C,), jnp.int32),
            pltpu.VMEM((C,), jnp.int32),
            pltpu.VMEM((C, D), jnp.float32),
            pltpu.VMEM((C, D), jnp.float32),
            pltpu.VMEM((C, FP), jnp.float32),
            pltpu.VMEM((C, FP), jnp.float32),
            pltpu.SemaphoreType.DMA,
            pltpu.SemaphoreType.DMA,
            pltpu.SemaphoreType.DMA,
            pltpu.SemaphoreType.DMA,
        ],
    )
    def k(s_hbm, f_hbm, src_hbm, dst_hbm, o_ss, o_ds, o_fs, o_fd,
          idx_s, idx_d, rows_s, rows_d, frows_s, frows_d,
          sem_a, sem_b, sem_c, sem_d):
        wid = lax.axis_index("s") * info.num_cores + lax.axis_index("c")
        base = wid * per_w

        def chunk(g, carry):
            off = base + g * C
            pltpu.sync_copy(src_hbm.at[pl.ds(off, C)], idx_s)
            pltpu.sync_copy(dst_hbm.at[pl.ds(off, C)], idx_d)
            c1 = pltpu.async_copy(s_hbm.at[idx_s], rows_s, sem_a)
            c2 = pltpu.async_copy(s_hbm.at[idx_d], rows_d, sem_b)
            c3 = pltpu.async_copy(f_hbm.at[idx_s], frows_s, sem_c)
            c4 = pltpu.async_copy(f_hbm.at[idx_d], frows_d, sem_d)
            c1.wait()
            c2.wait()
            c3.wait()
            c4.wait()
            s1 = pltpu.async_copy(rows_s, o_ss.at[pl.ds(off, C)], sem_a)
            s2 = pltpu.async_copy(rows_d, o_ds.at[pl.ds(off, C)], sem_b)
            s3 = pltpu.async_copy(frows_s, o_fs.at[pl.ds(off, C)], sem_c)
            s4 = pltpu.async_copy(frows_d, o_fd.at[pl.ds(off, C)], sem_d)
            s1.wait()
            s2.wait()
            s3.wait()
            s4.wait()
            return carry

        lax.fori_loop(0, n_chunks, chunk, 0)

    return k(s, feature, edge_src, edge_dst)


def _tc_body(seg_dst_ref, nseg_ref, src_ref, dst_ref, fsrc_ref, fdst_ref,
             dist_ref, lsi_ref, w1_ref, b1_ref, w2_ref, b2_ref, w3_ref,
             b3_ref, sg_ref, out_ref, num_ref, den_ref, part_ref):
    b = pl.program_id(0)

    @pl.when(b == 0)
    def _init():
        num_ref[...] = jnp.zeros_like(num_ref)
        den_ref[...] = jnp.zeros_like(den_ref)

    x = jnp.concatenate(
        [fsrc_ref[:, :H], fdst_ref[:, :H], dist_ref[...]], axis=1)
    h = 1.0 / (1.0 + jnp.exp(-(jnp.dot(x, w1_ref[...],
                                       preferred_element_type=jnp.float32)
                               + b1_ref[...])))
    h = 1.0 / (1.0 + jnp.exp(-(jnp.dot(h, w2_ref[...],
                                       preferred_element_type=jnp.float32)
                               + b2_ref[...])))
    w3d = lax.dot_general(h, w3_ref[...], (((1,), (0,)), ((), ())),
                          preferred_element_type=jnp.float32) + b3_ref[...]

    src_flat = src_ref[...]
    src3 = src_flat.reshape(BLK, BT, H)
    dst3 = dst_ref[...].reshape(BLK, BT, H)
    wt = w3d[:, :H, :]
    wb = w3d[:, H:, :]
    dn = (((2,), (1,)), ((0,), (0,)))
    alpha = (lax.dot_general(src3, wt, dn, preferred_element_type=jnp.float32)
             + lax.dot_general(dst3, wb, dn,
                               preferred_element_type=jnp.float32))
    alpha = jnp.where(alpha >= 0, alpha, 0.01 * alpha)
    ex = jnp.exp(jnp.minimum(alpha, 80.0)).reshape(BLK, D)

    payload = jnp.concatenate([ex * src_flat, ex], axis=1)

    lsi = jnp.broadcast_to(lsi_ref[0], (BLK, BLK))
    pt = (lax.broadcasted_iota(jnp.int32, (BLK, BLK), 0) == lsi)
    part_ref[...] = jnp.dot(pt.astype(jnp.float32), payload,
                            preferred_element_type=jnp.float32)

    ns = nseg_ref[b]

    def scatter(j, carry):
        d = seg_dst_ref[b * BLK + j]
        num_ref[pl.ds(d, 1), :] = (num_ref[pl.ds(d, 1), :]
                                   + part_ref[pl.ds(j, 1), :D])
        den_ref[pl.ds(d, 1), :] = (den_ref[pl.ds(d, 1), :]
                                   + part_ref[pl.ds(j, 1), D:])
        return carry

    lax.fori_loop(0, ns, scatter, 0)

    @pl.when(b == NBE - 1)
    def _final():
        den = den_ref[...]
        dsafe = jnp.where(den > 0, den, 1.0)
        out_ref[...] = jnp.maximum(num_ref[...] / dsafe, 0.0) * sg_ref[0, 0]


def _tc_compute(src_st, dst_st, fsrc, fdst, edge_dist, lsi, seg_dst, nseg,
                W1, b1, W2, b2, W3, b3, sg):
    grid_spec = pltpu.PrefetchScalarGridSpec(
        num_scalar_prefetch=2,
        grid=(NBE,),
        in_specs=[
            pl.BlockSpec((BLK, D), lambda b, *_: (b, 0)),
            pl.BlockSpec((BLK, D), lambda b, *_: (b, 0)),
            pl.BlockSpec((BLK, FP), lambda b, *_: (b, 0)),
            pl.BlockSpec((BLK, FP), lambda b, *_: (b, 0)),
            pl.BlockSpec((BLK, 1), lambda b, *_: (b, 0)),
            pl.BlockSpec((1, 1, BLK), lambda b, *_: (b, 0, 0)),
            pl.BlockSpec((2 * H + 1, 32), lambda b, *_: (0, 0)),
            pl.BlockSpec((1, 32), lambda b, *_: (0, 0)),
            pl.BlockSpec((32, 16), lambda b, *_: (0, 0)),
            pl.BlockSpec((1, 16), lambda b, *_: (0, 0)),
            pl.BlockSpec((16, 2 * H, H), lambda b, *_: (0, 0, 0)),
            pl.BlockSpec((1, 2 * H, H), lambda b, *_: (0, 0, 0)),
            pl.BlockSpec(memory_space=pltpu.SMEM),
        ],
        out_specs=pl.BlockSpec((N, D), lambda b, *_: (0, 0)),
        scratch_shapes=[
            pltpu.VMEM((N, D), jnp.float32),
            pltpu.VMEM((N, D), jnp.float32),
            pltpu.VMEM((BLK, 2 * D), jnp.float32),
        ],
    )
    return pl.pallas_call(
        _tc_body,
        grid_spec=grid_spec,
        out_shape=jax.ShapeDtypeStruct((N, D), jnp.float32),
    )(seg_dst, nseg, src_st, dst_st, fsrc, fdst, edge_dist, lsi,
      W1, b1, W2, b2, W3.reshape(16, 2 * H, H), b3.reshape(1, 2 * H, H), sg)


def kernel(state, feature, edge_dist, W1, b1, W2, b2, W3, b3, gate,
           edge_src, edge_dst):
    s = jnp.transpose(state, (2, 0, 1, 3)).reshape(N, D)

    feature_p = jnp.pad(feature, ((0, 0), (0, FP - H)))
    src_st, dst_st, fsrc, fdst = _sc_gather(s, feature_p, edge_src, edge_dst)

    prev = jnp.concatenate([jnp.full((1,), -1, jnp.int32), edge_dst[:-1]])
    pos = jnp.arange(E, dtype=jnp.int32) % BLK
    is_start = (edge_dst != prev) | (pos == 0)
    lsi = jnp.cumsum(is_start.reshape(NBE, BLK).astype(jnp.int32),
                     axis=1) - 1
    nseg = lsi[:, -1] + 1
    seg_dst = jnp.zeros((NBE, BLK), jnp.int32).at[
        jnp.arange(E, dtype=jnp.int32) // BLK, lsi.reshape(-1)
    ].set(edge_dst).reshape(-1)

    out2d = _tc_compute(
        src_st, dst_st, fsrc, fdst, edge_dist,
        lsi.reshape(NBE, 1, BLK), seg_dst, nseg,
        W1, b1.reshape(1, -1), W2, b2.reshape(1, -1), W3,
        b3.reshape(1, -1), jax.nn.sigmoid(gate))

    return jnp.transpose(out2d.reshape(N, B, T, H), (1, 2, 0, 3))
